# baseline (speedup 1.0000x reference)
import dataclasses
import functools

import jax
import jax.numpy as jnp
from jax import lax
from jax.experimental import pallas as pl
from jax.experimental.pallas import tpu as pltpu
from jax.experimental.pallas import tpu_sc as plsc

NUM_BLOCKS = 2048
BLOCK_SIZE = 256
D = NUM_BLOCKS * BLOCK_SIZE
KEY_DIM = 64
K = 8
SEED = 1234
EPS = 1e-08
T = 16384

OCC = T * K
NSUB = 16
SLICE = OCC // NSUB
NVEC = SLICE // 16
BATCH = 128
HALF = D // 2
CHROWS = 8576
NCH = 31
PAD_ROWS = 64
TOTROWS = CHROWS + PAD_ROWS
ROWW = 128
SELROWS = 104
NBIN = 48


def _hash_u32(x):
    x = x.astype(jnp.uint32)
    x = x ^ (x >> 16)
    x = x * jnp.uint32(0x7FEB352D)
    x = x ^ (x >> 15)
    x = x * jnp.uint32(0x846CA68B)
    x = x ^ (x >> 16)
    return x


def _prep_body(val_ref, v_ref):
    v = val_ref[...]
    norm = jnp.sqrt(jnp.sum(v * v, axis=1, keepdims=True)) + EPS
    vn = v / norm
    ones = jnp.ones((T, 16), jnp.float32)
    pad = jnp.zeros((T, ROWW - KEY_DIM - 16), jnp.float32)
    v_ref[...] = jnp.concatenate([vn, ones, pad], axis=1)


def _sc_body(hx_hbm, v_hbm, r_hbm, chunk_sp, hx_v, addr_v, sel_l, sel_o,
             tok_v, rows_v, hist_v, start_v, nbr_v, cur_v, dsem):
    c = lax.axis_index("c")
    s = lax.axis_index("s")
    base_occ = s * SLICE
    ntok = SLICE // K
    half_base = c * HALF

    pltpu.sync_copy(hx_hbm.at[pl.ds(s * ntok, ntok)], hx_v)

    @pl.loop(0, NVEC)
    def _haddr(i):
        lane = i * 16 + lax.iota(jnp.int32, 16)
        tok = lax.shift_right_logical(lane, 3)
        hxv = plsc.load_gather(hx_v, [tok])
        j = (lane & (K - 1)).astype(jnp.uint32)
        h = hxv.astype(jnp.uint32)
        hbase = _hash_u32(h ^ jnp.uint32(SEED))
        hj = _hash_u32(hbase + j * jnp.uint32(0x9E3779B1))
        block = (hj & jnp.uint32(NUM_BLOCKS - 1)).astype(jnp.int32)
        off = (_hash_u32(hj ^ jnp.uint32(0x85EBCA6B))
               & jnp.uint32(BLOCK_SIZE - 1)).astype(jnp.int32)
        addr_v[pl.ds(i * 16, 16)] = block * BLOCK_SIZE + off

    @pl.loop(0, BATCH)
    def _zinit(r):
        for cg in range(ROWW // 16):
            rows_v[r, pl.ds(cg * 16, 16)] = jnp.zeros((16,), jnp.float32)

    for g in range(NBIN // 16):
        hist_v[pl.ds(g * 16, 16)] = jnp.zeros((16,), jnp.int32)

    @pl.loop(0, SELROWS)
    def _pf(r):
        for cg in range(BATCH // 16):
            colv = cg * 16 + lax.iota(jnp.int32, 16)
            sl = pl.ds(cg * 16, 16)
            sel_l[r, sl] = jnp.full((16,), CHROWS, jnp.int32) + (colv & (PAD_ROWS - 1))
            sel_o[r, sl] = jnp.full((16,), OCC, jnp.int32) + colv

    @pl.loop(0, NVEC)
    def _hist(i):
        av = addr_v[pl.ds(i * 16, 16)]
        la = av - half_base
        inhalf = (la >= 0) & (la < HALF)
        ci = jnp.where(inhalf, lax.div(la, CHROWS), NCH)
        rank, last = plsc.scan_count(ci)
        plsc.addupdate_scatter(hist_v, [ci], rank, mask=last)

    carry = jnp.zeros((16,), jnp.int32)
    for g in range(NBIN // 16):
        hg = hist_v[pl.ds(g * 16, 16)]
        nbg = lax.shift_right_logical(hg + (BATCH - 1), 7)
        incl = plsc.cumsum(nbg)
        excl = incl - nbg + carry
        start_v[pl.ds(g * 16, 16)] = excl
        nbr_v[pl.ds(g * 16, 16)] = nbg
        cur_v[pl.ds(g * 16, 16)] = excl * BATCH
        carry = carry + jnp.max(incl)

    @pl.loop(0, NVEC)
    def _place(i):
        av = addr_v[pl.ds(i * 16, 16)]
        la = av - half_base
        inhalf = (la >= 0) & (la < HALF)
        ci = jnp.where(inhalf, lax.div(la, CHROWS), NCH)
        rank, last = plsc.scan_count(ci)
        basew = plsc.load_gather(cur_v, [ci])
        pos = basew + rank - 1
        row = lax.shift_right_logical(pos, 7)
        col = pos & (BATCH - 1)
        occ = base_occ + i * 16 + lax.iota(jnp.int32, 16)
        local = la - ci * CHROWS
        plsc.store_scatter(sel_l, [row, col], local, mask=inhalf)
        plsc.store_scatter(sel_o, [row, col], occ, mask=inhalf)
        plsc.addupdate_scatter(cur_v, [ci], rank, mask=last)

    @pl.loop(0, NCH)
    def _chunk(cc):
        ccv = jnp.full((16,), 0, jnp.int32) + cc
        sr = jnp.max(plsc.load_gather(start_v, [ccv]))
        nb = jnp.max(plsc.load_gather(nbr_v, [ccv]))

        @pl.loop(0, nb)
        def _zero(b):
            pltpu.async_copy(rows_v, chunk_sp.at[sel_l.at[sr + b]], dsem)

        @pl.loop(0, nb)
        def _zdrain(b):
            pltpu.make_async_copy(rows_v, chunk_sp.at[sel_l.at[sr + b]], dsem).wait()

        plsc.subcore_barrier()

        @pl.loop(0, nb)
        def _add(b):
            for cg in range(BATCH // 16):
                sl = pl.ds(cg * 16, 16)
                tok_v[0, sl] = lax.shift_right_logical(sel_o[sr + b, sl], 3)
            pltpu.sync_copy(v_hbm.at[tok_v.at[0]], rows_v)
            pltpu.sync_copy(rows_v, chunk_sp.at[sel_l.at[sr + b]], add=True)

        plsc.subcore_barrier()

        @pl.loop(0, nb)
        def _read(b):
            pltpu.sync_copy(chunk_sp.at[sel_l.at[sr + b]], rows_v)
            pltpu.sync_copy(rows_v, r_hbm.at[sel_o.at[sr + b]])

        @pl.when(nb > 0)
        def _rz_all():
            @pl.loop(0, BATCH)
            def _rz(r):
                for cg in range(ROWW // 16):
                    rows_v[r, pl.ds(cg * 16, 16)] = jnp.zeros((16,), jnp.float32)

        plsc.subcore_barrier()


TB = 2048


def _mean_body(r_ref, out_ref):
    x = r_ref[...]
    xr = x.reshape(TB, K, ROWW)
    cnt = jnp.maximum(xr[:, :, KEY_DIM:KEY_DIM + 1], 1.0)
    out_ref[...] = jnp.sum(xr[:, :, :KEY_DIM] / cnt, axis=1) * (1.0 / K)


@jax.jit
def _run(hx_flat, values):
    v128 = pl.pallas_call(
        _prep_body,
        in_specs=[pl.BlockSpec((T, KEY_DIM), lambda: (0, 0))],
        out_specs=pl.BlockSpec((T, ROWW), lambda: (0, 0)),
        out_shape=jax.ShapeDtypeStruct((T, ROWW), jnp.float32),
    )(values)

    mesh = plsc.VectorSubcoreMesh(core_axis_name="c", subcore_axis_name="s")
    cp = pltpu.CompilerParams()
    if "needs_layout_passes" in pltpu.CompilerParams.__dataclass_fields__:
        cp = dataclasses.replace(cp, needs_layout_passes=False)
    r = pl.kernel(
        _sc_body,
        out_type=jax.ShapeDtypeStruct((OCC + BATCH, ROWW), jnp.float32),
        mesh=mesh,
        compiler_params=cp,
        scratch_types=[
            pltpu.VMEM_SHARED((TOTROWS, ROWW), jnp.float32),
            pltpu.VMEM((SLICE // K,), jnp.int32),
            pltpu.VMEM((SLICE,), jnp.int32),
            pltpu.VMEM((SELROWS, BATCH), jnp.int32),
            pltpu.VMEM((SELROWS, BATCH), jnp.int32),
            pltpu.VMEM((1, BATCH), jnp.int32),
            pltpu.VMEM((BATCH, ROWW), jnp.float32),
            pltpu.VMEM((NBIN,), jnp.int32),
            pltpu.VMEM((NBIN,), jnp.int32),
            pltpu.VMEM((NBIN,), jnp.int32),
            pltpu.VMEM((NBIN,), jnp.int32),
            pltpu.SemaphoreType.DMA,
        ],
    )(hx_flat, v128)

    out = pl.pallas_call(
        _mean_body,
        grid=(T // TB,),
        in_specs=[pl.BlockSpec((TB * K, ROWW), lambda i: (i, 0))],
        out_specs=pl.BlockSpec((TB, KEY_DIM), lambda i: (i, 0)),
        out_shape=jax.ShapeDtypeStruct((T, KEY_DIM), jnp.float32),
    )(r)
    return out


def kernel(memory, counts, hx_tensor, values):
    del memory, counts
    hx_flat = hx_tensor.reshape(T).astype(jnp.int32)
    return _run(hx_flat, values)


# speedup vs baseline: 1.4182x; 1.4182x over previous
import dataclasses
import functools

import jax
import jax.numpy as jnp
from jax import lax
from jax.experimental import pallas as pl
from jax.experimental.pallas import tpu as pltpu
from jax.experimental.pallas import tpu_sc as plsc

NUM_BLOCKS = 2048
BLOCK_SIZE = 256
D = NUM_BLOCKS * BLOCK_SIZE
KEY_DIM = 64
K = 8
SEED = 1234
EPS = 1e-08
T = 16384

OCC = T * K
NSUB = 16
SLICE = OCC // NSUB
NVEC = SLICE // 16
BATCH = 128
HALF = D // 2
CHROWS = 7040
NCH = 38
PAD_ROWS = 64
TOTROWS = CHROWS + PAD_ROWS
ROWW = 128
SELROWS = 104
NBIN = 48


def _hash_u32(x):
    x = x.astype(jnp.uint32)
    x = x ^ (x >> 16)
    x = x * jnp.uint32(0x7FEB352D)
    x = x ^ (x >> 15)
    x = x * jnp.uint32(0x846CA68B)
    x = x ^ (x >> 16)
    return x


def _prep_body(val_ref, v_ref):
    v = val_ref[...]
    norm = jnp.sqrt(jnp.sum(v * v, axis=1, keepdims=True)) + EPS
    vn = v / norm
    ones = jnp.ones((T, 16), jnp.float32)
    pad = jnp.zeros((T, ROWW - KEY_DIM - 16), jnp.float32)
    v_ref[...] = jnp.concatenate([vn, ones, pad], axis=1)


def _sc_body(hx_hbm, v_hbm, r_hbm, chunk_sp, hx_v, addr_v, sel_l, sel_t, sel_o,
             rows_v, hist_v, start_v, nbr_v, cur_v):
    c = lax.axis_index("c")
    s = lax.axis_index("s")
    base_occ = s * SLICE
    ntok = SLICE // K
    half_base = c * HALF

    pltpu.sync_copy(hx_hbm.at[pl.ds(s * ntok, ntok)], hx_v)

    @pl.loop(0, NVEC)
    def _haddr(i):
        lane = i * 16 + lax.iota(jnp.int32, 16)
        tok = lax.shift_right_logical(lane, 3)
        hxv = plsc.load_gather(hx_v, [tok])
        j = (lane & (K - 1)).astype(jnp.uint32)
        h = hxv.astype(jnp.uint32)
        hbase = _hash_u32(h ^ jnp.uint32(SEED))
        hj = _hash_u32(hbase + j * jnp.uint32(0x9E3779B1))
        block = (hj & jnp.uint32(NUM_BLOCKS - 1)).astype(jnp.int32)
        off = (_hash_u32(hj ^ jnp.uint32(0x85EBCA6B))
               & jnp.uint32(BLOCK_SIZE - 1)).astype(jnp.int32)
        addr_v[pl.ds(i * 16, 16)] = block * BLOCK_SIZE + off

    @pl.loop(0, BATCH)
    def _zinit(r):
        for cg in range(ROWW // 16):
            rows_v[r, pl.ds(cg * 16, 16)] = jnp.zeros((16,), jnp.float32)

    for g in range(NBIN // 16):
        hist_v[pl.ds(g * 16, 16)] = jnp.zeros((16,), jnp.int32)

    @pl.loop(0, SELROWS)
    def _pf(r):
        for cg in range(BATCH // 16):
            colv = cg * 16 + lax.iota(jnp.int32, 16)
            sl = pl.ds(cg * 16, 16)
            sel_l[r, sl] = jnp.full((16,), CHROWS, jnp.int32) + (colv & (PAD_ROWS - 1))
            sel_t[r, sl] = (r * BATCH + colv) & (T // 2 - 1)
            sel_o[r, sl] = jnp.full((16,), OCC, jnp.int32) + colv

    @pl.loop(0, NVEC)
    def _hist(i):
        av = addr_v[pl.ds(i * 16, 16)]
        la = av - half_base
        inhalf = (la >= 0) & (la < HALF)
        ci = jnp.where(inhalf, lax.div(la, CHROWS), NCH)
        rank, last = plsc.scan_count(ci)
        plsc.addupdate_scatter(hist_v, [ci], rank, mask=last)

    carry = jnp.zeros((16,), jnp.int32)
    for g in range(NBIN // 16):
        hg = hist_v[pl.ds(g * 16, 16)]
        nbg = lax.shift_right_logical(hg + (BATCH - 1), 7)
        incl = plsc.cumsum(nbg)
        excl = incl - nbg + carry
        start_v[pl.ds(g * 16, 16)] = excl
        nbr_v[pl.ds(g * 16, 16)] = nbg
        cur_v[pl.ds(g * 16, 16)] = excl * BATCH
        carry = carry + jnp.max(incl)

    @pl.loop(0, NVEC)
    def _place(i):
        av = addr_v[pl.ds(i * 16, 16)]
        la = av - half_base
        inhalf = (la >= 0) & (la < HALF)
        ci = jnp.where(inhalf, lax.div(la, CHROWS), NCH)
        rank, last = plsc.scan_count(ci)
        basew = plsc.load_gather(cur_v, [ci])
        pos = basew + rank - 1
        row = lax.shift_right_logical(pos, 7)
        col = pos & (BATCH - 1)
        occ = base_occ + i * 16 + lax.iota(jnp.int32, 16)
        local = la - ci * CHROWS
        plsc.store_scatter(sel_l, [row, col], local, mask=inhalf)
        plsc.store_scatter(sel_t, [row, col],
                           lax.shift_right_logical(occ, 3), mask=inhalf)
        plsc.store_scatter(sel_o, [row, col], occ, mask=inhalf)
        plsc.addupdate_scatter(cur_v, [ci], rank, mask=last)

    @pl.loop(0, NCH)
    def _chunk(cc):
        ccv = jnp.full((16,), 0, jnp.int32) + cc
        sr = jnp.max(plsc.load_gather(start_v, [ccv]))
        nb = jnp.max(plsc.load_gather(nbr_v, [ccv]))

        @pl.loop(0, nb)
        def _zero(b):
            pltpu.sync_copy(rows_v, chunk_sp.at[sel_l.at[sr + b]])

        plsc.subcore_barrier()

        @pl.loop(0, nb)
        def _add(b):
            pltpu.sync_copy(v_hbm.at[sel_t.at[sr + b]], rows_v)
            pltpu.sync_copy(rows_v, chunk_sp.at[sel_l.at[sr + b]], add=True)

        plsc.subcore_barrier()

        @pl.loop(0, nb)
        def _read(b):
            pltpu.sync_copy(chunk_sp.at[sel_l.at[sr + b]], rows_v)
            pltpu.sync_copy(rows_v, r_hbm.at[sel_o.at[sr + b]])

        @pl.when(nb > 0)
        def _rz_all():
            @pl.loop(0, BATCH)
            def _rz(r):
                for cg in range(ROWW // 16):
                    rows_v[r, pl.ds(cg * 16, 16)] = jnp.zeros((16,), jnp.float32)

        plsc.subcore_barrier()


TB = 2048


def _mean_body(r_ref, out_ref):
    x = r_ref[...]
    xr = x.reshape(TB, K, ROWW)
    cnt = jnp.maximum(xr[:, :, KEY_DIM:KEY_DIM + 1], 1.0)
    out_ref[...] = jnp.sum(xr[:, :, :KEY_DIM] / cnt, axis=1) * (1.0 / K)


@jax.jit
def _run(hx_flat, values):
    v128 = pl.pallas_call(
        _prep_body,
        in_specs=[pl.BlockSpec((T, KEY_DIM), lambda: (0, 0))],
        out_specs=pl.BlockSpec((T, ROWW), lambda: (0, 0)),
        out_shape=jax.ShapeDtypeStruct((T, ROWW), jnp.float32),
    )(values)

    mesh = plsc.VectorSubcoreMesh(core_axis_name="c", subcore_axis_name="s")
    cp = pltpu.CompilerParams()
    if "needs_layout_passes" in pltpu.CompilerParams.__dataclass_fields__:
        cp = dataclasses.replace(cp, needs_layout_passes=False)
    r = pl.kernel(
        _sc_body,
        out_type=jax.ShapeDtypeStruct((OCC + BATCH, ROWW), jnp.float32),
        mesh=mesh,
        compiler_params=cp,
        scratch_types=[
            pltpu.VMEM_SHARED((TOTROWS, ROWW), jnp.float32),
            pltpu.VMEM((SLICE // K,), jnp.int32),
            pltpu.VMEM((SLICE,), jnp.int32),
            pltpu.VMEM((SELROWS, BATCH), jnp.int32),
            pltpu.VMEM((SELROWS, BATCH), jnp.int32),
            pltpu.VMEM((SELROWS, BATCH), jnp.int32),
            pltpu.VMEM((BATCH, ROWW), jnp.float32),
            pltpu.VMEM((NBIN,), jnp.int32),
            pltpu.VMEM((NBIN,), jnp.int32),
            pltpu.VMEM((NBIN,), jnp.int32),
            pltpu.VMEM((NBIN,), jnp.int32),
        ],
    )(hx_flat, v128)

    out = pl.pallas_call(
        _mean_body,
        grid=(T // TB,),
        in_specs=[pl.BlockSpec((TB * K, ROWW), lambda i: (i, 0))],
        out_specs=pl.BlockSpec((TB, KEY_DIM), lambda i: (i, 0)),
        out_shape=jax.ShapeDtypeStruct((T, KEY_DIM), jnp.float32),
    )(r)
    return out


def kernel(memory, counts, hx_tensor, values):
    del memory, counts
    hx_flat = hx_tensor.reshape(T).astype(jnp.int32)
    return _run(hx_flat, values)
